# baseline (speedup 1.0000x reference)
import functools

import jax
import jax.numpy as jnp
from jax import lax
from jax.experimental import pallas as pl
from jax.experimental.pallas import tpu as pltpu
from jax.experimental.pallas import tpu_sc as plsc

BS, L, DIM = 4, 576, 768
INNER = DIM * 2
G, K, VD = 2, 320, 128
T = BS * L
R = T * G
TM = 256
NB = T // TM


def _mlp_argmax_body(x_ref, w1_ref, b1_ref, w2_ref, b2_ref, g_ref, idx_ref):
    h = jnp.dot(x_ref[...], w1_ref[...], preferred_element_type=jnp.float32)
    h = jax.nn.gelu(h + b1_ref[...])
    logits = jnp.dot(h, w2_ref[...], preferred_element_type=jnp.float32)
    s = logits + b2_ref[...]
    gum = g_ref[...]
    riota = lax.broadcasted_iota(jnp.int32, (TM, G * TM), 0)
    ciota = lax.broadcasted_iota(jnp.int32, (TM, G * TM), 1)
    for g in range(G):
        sel = (ciota == G * riota + g).astype(jnp.float32)
        gum_g = jnp.dot(sel, gum, preferred_element_type=jnp.float32,
                        precision=lax.Precision.HIGHEST)
        sg = s[:, g * K:(g + 1) * K] + gum_g
        m = jnp.max(sg, axis=1, keepdims=True)
        iota = lax.broadcasted_iota(jnp.int32, sg.shape, 1)
        idx_ref[0, g, :] = jnp.min(jnp.where(sg == m, iota, K), axis=1)


NC, NS = 2, 16
NW = NC * NS
NU = R // 128


def _gather_unit(u, table_hbm, idx4_hbm, out_hbm, tmp_v, qidx_v, rows_v, sem):
    s = u >> 2
    tc = (u >> 1) & 1
    h = u & 1
    pltpu.sync_copy(idx4_hbm.at[s, tc, 0, pl.ds(h * 64, 64)],
                    tmp_v.at[pl.ds(0, 64)])
    pltpu.sync_copy(idx4_hbm.at[s, tc, 1, pl.ds(h * 64, 64)],
                    tmp_v.at[pl.ds(64, 64)])
    lane = lax.broadcasted_iota(jnp.int32, (16,), 0)
    for c in range(8):
        a = tmp_v[pl.ds(8 * c, 16)]
        b = tmp_v[pl.ds(56 + 8 * c, 16)]
        qidx_v[pl.ds(c * 16, 16)] = jnp.where(lane < 8, a, b)
    pltpu.async_copy(table_hbm.at[qidx_v], rows_v, sem).wait()
    pltpu.sync_copy(rows_v, out_hbm.at[pl.ds(u * 128, 128)])


def _gather_body(table_hbm, idx4_hbm, out_hbm, tmp_v, qidx_v, rows_v, sem):
    wid = lax.axis_index("s") * NC + lax.axis_index("c")
    _gather_unit(wid, table_hbm, idx4_hbm, out_hbm,
                 tmp_v, qidx_v, rows_v, sem)

    @pl.when(wid + NW < NU)
    def _():
        _gather_unit(wid + NW, table_hbm, idx4_hbm, out_hbm,
                     tmp_v, qidx_v, rows_v, sem)


@functools.cache
def _sc_gather():
    return functools.partial(
        pl.kernel,
        mesh=plsc.VectorSubcoreMesh(core_axis_name="c", subcore_axis_name="s"),
        out_type=jax.ShapeDtypeStruct((R, VD), jnp.float32),
        scratch_types=[
            pltpu.VMEM((128,), jnp.int32),
            pltpu.VMEM((128,), jnp.int32),
            pltpu.VMEM((128, VD), jnp.float32),
            pltpu.SemaphoreType.DMA,
        ],
    )(_gather_body)


def kernel(x, W1, b1, W2, b2, embeddings, gumbels):
    x2 = x.reshape(T, DIM)
    idxp = pl.pallas_call(
        _mlp_argmax_body,
        grid=(NB,),
        in_specs=[
            pl.BlockSpec((TM, DIM), lambda i: (i, 0)),
            pl.BlockSpec((DIM, INNER), lambda i: (0, 0)),
            pl.BlockSpec((1, INNER), lambda i: (0, 0)),
            pl.BlockSpec((INNER, G * K), lambda i: (0, 0)),
            pl.BlockSpec((1, G * K), lambda i: (0, 0)),
            pl.BlockSpec((G * TM, K), lambda i: (i, 0)),
        ],
        out_specs=pl.BlockSpec((1, 8, TM), lambda i: (i, 0, 0)),
        out_shape=jax.ShapeDtypeStruct((NB, 8, TM), jnp.int32),
    )(x2, W1, b1.reshape(1, INNER), W2, b2.reshape(1, G * K), gumbels)
    table = embeddings.reshape(K, VD)
    idx4 = idxp.reshape(NB, 8, 2, 128).transpose(0, 2, 1, 3)
    out = _sc_gather()(table, idx4)
    return (out.reshape(T // 8, G, 8, VD)
               .transpose(0, 2, 1, 3)
               .reshape(BS, L, G * VD))


# speedup vs baseline: 1.1496x; 1.1496x over previous
import functools

import jax
import jax.numpy as jnp
from jax import lax
from jax.experimental import pallas as pl
from jax.experimental.pallas import tpu as pltpu
from jax.experimental.pallas import tpu_sc as plsc

BS, L, DIM = 4, 576, 768
INNER = DIM * 2
G, K, VD = 2, 320, 128
T = BS * L
R = T * G
TM = 256
NB = T // TM


def _mlp_argmax_body(x_ref, w1_ref, b1_ref, w2_ref, b2_ref, g_ref, idx_ref):
    h = jnp.dot(x_ref[...], w1_ref[...], preferred_element_type=jnp.float32)
    h = jax.nn.gelu(h + b1_ref[...])
    logits = jnp.dot(h, w2_ref[...], preferred_element_type=jnp.float32)
    s = logits + b2_ref[...]
    gum = g_ref[...].reshape(TM, G, K)
    for g in range(G):
        sg = s[:, g * K:(g + 1) * K] + gum[:, g, :]
        m = jnp.max(sg, axis=1, keepdims=True)
        iota = lax.broadcasted_iota(jnp.int32, sg.shape, 1)
        idx_ref[0, g, :] = jnp.min(jnp.where(sg == m, iota, K), axis=1)


NC, NS = 2, 16
NW = NC * NS
NU = R // 128


def _gather_unit(u, table_hbm, idx4_hbm, out_hbm, tmp_v, qidx_v, rows_v, sem):
    s = u >> 2
    tc = (u >> 1) & 1
    h = u & 1
    pltpu.sync_copy(idx4_hbm.at[s, tc, 0, pl.ds(h * 64, 64)],
                    tmp_v.at[pl.ds(0, 64)])
    pltpu.sync_copy(idx4_hbm.at[s, tc, 1, pl.ds(h * 64, 64)],
                    tmp_v.at[pl.ds(64, 64)])
    lane = lax.broadcasted_iota(jnp.int32, (16,), 0)
    for c in range(8):
        a = tmp_v[pl.ds(8 * c, 16)]
        b = tmp_v[pl.ds(56 + 8 * c, 16)]
        qidx_v[pl.ds(c * 16, 16)] = jnp.where(lane < 8, a, b)
    pltpu.async_copy(table_hbm.at[qidx_v], rows_v, sem).wait()
    pltpu.sync_copy(rows_v, out_hbm.at[pl.ds(u * 128, 128)])


def _gather_body(table_hbm, idx4_hbm, out_hbm, tmp_v, qidx_v, rows_v, sem):
    wid = lax.axis_index("s") * NC + lax.axis_index("c")
    _gather_unit(wid, table_hbm, idx4_hbm, out_hbm,
                 tmp_v, qidx_v, rows_v, sem)

    @pl.when(wid + NW < NU)
    def _():
        _gather_unit(wid + NW, table_hbm, idx4_hbm, out_hbm,
                     tmp_v, qidx_v, rows_v, sem)


@functools.cache
def _sc_gather():
    return functools.partial(
        pl.kernel,
        mesh=plsc.VectorSubcoreMesh(core_axis_name="c", subcore_axis_name="s"),
        out_type=jax.ShapeDtypeStruct((R, VD), jnp.float32),
        scratch_types=[
            pltpu.VMEM((128,), jnp.int32),
            pltpu.VMEM((128,), jnp.int32),
            pltpu.VMEM((128, VD), jnp.float32),
            pltpu.SemaphoreType.DMA,
        ],
    )(_gather_body)


def kernel(x, W1, b1, W2, b2, embeddings, gumbels):
    x2 = x.reshape(T, DIM)
    idxp = pl.pallas_call(
        _mlp_argmax_body,
        grid=(NB,),
        in_specs=[
            pl.BlockSpec((TM, DIM), lambda i: (i, 0)),
            pl.BlockSpec((DIM, INNER), lambda i: (0, 0)),
            pl.BlockSpec((1, INNER), lambda i: (0, 0)),
            pl.BlockSpec((INNER, G * K), lambda i: (0, 0)),
            pl.BlockSpec((1, G * K), lambda i: (0, 0)),
            pl.BlockSpec((G * TM, K), lambda i: (i, 0)),
        ],
        out_specs=pl.BlockSpec((1, 8, TM), lambda i: (i, 0, 0)),
        out_shape=jax.ShapeDtypeStruct((NB, 8, TM), jnp.int32),
    )(x2, W1, b1.reshape(1, INNER), W2, b2.reshape(1, G * K), gumbels)
    table = embeddings.reshape(K, VD)
    idx4 = idxp.reshape(NB, 8, 2, 128).transpose(0, 2, 1, 3)
    out = _sc_gather()(table, idx4)
    return (out.reshape(T // 8, G, 8, VD)
               .transpose(0, 2, 1, 3)
               .reshape(BS, L, G * VD))


# speedup vs baseline: 1.2066x; 1.0495x over previous
import functools

import jax
import jax.numpy as jnp
from jax import lax
from jax.experimental import pallas as pl
from jax.experimental.pallas import tpu as pltpu
from jax.experimental.pallas import tpu_sc as plsc

BS, L, DIM = 4, 576, 768
INNER = DIM * 2
G, K, VD = 2, 320, 128
T = BS * L
R = T * G
TM = 768
NB = T // TM


def _mlp_argmax_body(x_ref, w1_ref, b1_ref, w2_ref, b2_ref, g_ref, idx_ref):
    h = jnp.dot(x_ref[...], w1_ref[...], preferred_element_type=jnp.float32)
    h = jax.nn.gelu(h + b1_ref[...])
    logits = jnp.dot(h, w2_ref[...], preferred_element_type=jnp.float32)
    s = logits + b2_ref[...]
    gum = g_ref[...].reshape(TM, G, K)
    for g in range(G):
        sg = s[:, g * K:(g + 1) * K] + gum[:, g, :]
        m = jnp.max(sg, axis=1, keepdims=True)
        iota = lax.broadcasted_iota(jnp.int32, sg.shape, 1)
        idx_ref[0, g, :] = jnp.min(jnp.where(sg == m, iota, K), axis=1)


NC, NS = 2, 16
NW = NC * NS
NU = R // 128


def _gather_unit(u, table_hbm, idx4_hbm, out_hbm, tmp_v, qidx_v, rows_v, sem):
    s = (u * 43691) >> 19
    tc = (u - 12 * s) >> 1
    h = u & 1
    pltpu.sync_copy(idx4_hbm.at[s, tc, 0, pl.ds(h * 64, 64)],
                    tmp_v.at[pl.ds(0, 64)])
    pltpu.sync_copy(idx4_hbm.at[s, tc, 1, pl.ds(h * 64, 64)],
                    tmp_v.at[pl.ds(64, 64)])
    lane = lax.broadcasted_iota(jnp.int32, (16,), 0)
    for c in range(8):
        a = tmp_v[pl.ds(8 * c, 16)]
        b = tmp_v[pl.ds(56 + 8 * c, 16)]
        qidx_v[pl.ds(c * 16, 16)] = jnp.where(lane < 8, a, b)
    pltpu.async_copy(table_hbm.at[qidx_v], rows_v, sem).wait()
    pltpu.sync_copy(rows_v, out_hbm.at[pl.ds(u * 128, 128)])


def _gather_body(table_hbm, idx4_hbm, out_hbm, tmp_v, qidx_v, rows_v, sem):
    wid = lax.axis_index("s") * NC + lax.axis_index("c")
    _gather_unit(wid, table_hbm, idx4_hbm, out_hbm,
                 tmp_v, qidx_v, rows_v, sem)

    @pl.when(wid + NW < NU)
    def _():
        _gather_unit(wid + NW, table_hbm, idx4_hbm, out_hbm,
                     tmp_v, qidx_v, rows_v, sem)


@functools.cache
def _sc_gather():
    return functools.partial(
        pl.kernel,
        mesh=plsc.VectorSubcoreMesh(core_axis_name="c", subcore_axis_name="s"),
        out_type=jax.ShapeDtypeStruct((R, VD), jnp.float32),
        scratch_types=[
            pltpu.VMEM((128,), jnp.int32),
            pltpu.VMEM((128,), jnp.int32),
            pltpu.VMEM((128, VD), jnp.float32),
            pltpu.SemaphoreType.DMA,
        ],
    )(_gather_body)


def kernel(x, W1, b1, W2, b2, embeddings, gumbels):
    x2 = x.reshape(T, DIM)
    idxp = pl.pallas_call(
        _mlp_argmax_body,
        grid=(NB,),
        in_specs=[
            pl.BlockSpec((TM, DIM), lambda i: (i, 0)),
            pl.BlockSpec((DIM, INNER), lambda i: (0, 0)),
            pl.BlockSpec((1, INNER), lambda i: (0, 0)),
            pl.BlockSpec((INNER, G * K), lambda i: (0, 0)),
            pl.BlockSpec((1, G * K), lambda i: (0, 0)),
            pl.BlockSpec((G * TM, K), lambda i: (i, 0)),
        ],
        out_specs=pl.BlockSpec((1, 8, TM), lambda i: (i, 0, 0)),
        out_shape=jax.ShapeDtypeStruct((NB, 8, TM), jnp.int32),
    )(x2, W1, b1.reshape(1, INNER), W2, b2.reshape(1, G * K), gumbels)
    table = embeddings.reshape(K, VD)
    idx4 = idxp.reshape(NB, 8, TM // 128, 128).transpose(0, 2, 1, 3)
    out = _sc_gather()(table, idx4)
    return (out.reshape(T // 8, G, 8, VD)
               .transpose(0, 2, 1, 3)
               .reshape(BS, L, G * VD))
